# baseline (speedup 1.0000x reference)
import numpy as np
import jax
import jax.numpy as jnp
from jax.experimental import pallas as pl
from jax.experimental.pallas import tpu as pltpu

_CDT = jnp.bfloat16


def _w1_sel():
    s = np.zeros((5, 5, 6, 28, 4, 12), np.float32)
    for rp in range(2):
        for wp in range(2):
            g = rp * 2 + wp
            for i in range(5):
                for j in range(5):
                    for ow in range(12):
                        s[i, j, i + rp, 2 * ow + wp + j, g, ow] = 1.0
    return s.reshape(25, 6 * 28 * 4 * 12)


def _w2_sel():
    s = np.zeros((5, 12, 2, 4), np.float32)
    for wp in range(2):
        for j in range(5):
            for o2 in range(4):
                s[j, 2 * o2 + wp + j, wp, o2] = 1.0
    return s.reshape(5, 96)


_W1_SEL = _w1_sel()
_W2_SEL = _w2_sel()


def _net_kernel(x_ref, w1_ref, b1_ref, w2_ref, b2_ref, fs_ref, fb1_ref,
                fw2_ref, fb2_ref, o_ref):
    G = o_ref.shape[0]
    f32 = jnp.float32
    xb = x_ref[...]

    w1 = w1_ref[...]
    b1 = b1_ref[...]
    pieces = []
    for hp in range(12):
        c1 = jnp.dot(xb[:, 56 * hp:56 * hp + 168], w1,
                     preferred_element_type=f32)
        m = jnp.maximum(jnp.maximum(c1[:, 0:256], c1[:, 256:512]),
                        jnp.maximum(c1[:, 512:768], c1[:, 768:1024]))
        y = jnp.maximum(m + b1, 0.0)
        pieces.append(y.astype(_CDT))
    y1 = jnp.concatenate(pieces, axis=1)

    w2 = w2_ref[...]
    b2 = b2_ref[...]
    pm = []
    for oh in range(8):
        c2 = jnp.dot(y1[:, 256 * oh:256 * oh + 1280], w2,
                     preferred_element_type=f32)
        pm.append(jnp.maximum(c2[:, 0:256], c2[:, 256:512]))

    h = jnp.zeros((G, 512), f32)
    for r in range(4):
        y2 = jnp.maximum(jnp.maximum(pm[2 * r], pm[2 * r + 1]) + b2, 0.0)
        h = h + jnp.dot(y2.astype(_CDT), fs_ref[r],
                        preferred_element_type=f32)

    hr = jnp.maximum(h + fb1_ref[...], 0.0).astype(_CDT)
    z = jnp.dot(hr, fw2_ref[...], preferred_element_type=f32) + fb2_ref[...]
    mz = jnp.max(z, axis=-1, keepdims=True)
    ez = jnp.exp(z - mz)
    lse = jnp.log(jnp.sum(ez, axis=-1, keepdims=True)) + mz
    o_ref[...] = z - lse


def kernel(x, w1, b1, w2, b2, se1, so1, s2, fc1w, fc1b, fc2w, fc2b):
    del se1, so1, s2
    B = x.shape[0]
    G = 256
    while B % G:
        G //= 2

    w1all = jnp.einsum("tc,tm->mc", w1[:, :20], _W1_SEL)
    w1all = w1all.reshape(168, 4, 240)
    w1all = jnp.pad(w1all, ((0, 0), (0, 0), (0, 16)))
    w1all = w1all.reshape(168, 1024).astype(_CDT)
    b1r = jnp.pad(jnp.tile(b1[0, :20], 12), (0, 16)).reshape(1, 256)
    w2c = w2[:, :20, :50].reshape(5, 5, 20, 50)
    w2s = jnp.einsum("ijab,jm->imab", w2c, _W2_SEL)
    w2s = w2s.reshape(5, 12, 2, 4, 20, 50).transpose(0, 1, 4, 2, 3, 5)
    w2s = w2s.reshape(5, 240, 2, 200)
    w2s = jnp.pad(w2s, ((0, 0), (0, 16), (0, 0), (0, 56)))
    w2big = w2s.reshape(1280, 512).astype(_CDT)
    b2r = jnp.pad(jnp.tile(b2[0, :50], 4), (0, 56)).reshape(1, 256)

    f3 = fc1w.reshape(16, 128, 512)[:, :50, :].reshape(4, 200, 512)
    fs = jnp.pad(f3, ((0, 0), (0, 56), (0, 0))).astype(_CDT)

    out = pl.pallas_call(
        _net_kernel,
        grid=(B // G,),
        out_shape=jax.ShapeDtypeStruct((B, 128), jnp.float32),
        in_specs=[
            pl.BlockSpec((G, 784), lambda b: (b, 0)),
            pl.BlockSpec((168, 1024), lambda b: (0, 0)),
            pl.BlockSpec((1, 256), lambda b: (0, 0)),
            pl.BlockSpec((1280, 512), lambda b: (0, 0)),
            pl.BlockSpec((1, 256), lambda b: (0, 0)),
            pl.BlockSpec((4, 256, 512), lambda b: (0, 0, 0)),
            pl.BlockSpec((1, 512), lambda b: (0, 0)),
            pl.BlockSpec((512, 128), lambda b: (0, 0)),
            pl.BlockSpec((1, 128), lambda b: (0, 0)),
        ],
        out_specs=pl.BlockSpec((G, 128), lambda b: (b, 0)),
        compiler_params=pltpu.CompilerParams(
            dimension_semantics=("parallel",)),
    )(x.reshape(B, 784).astype(_CDT), w1all, b1r, w2big, b2r, fs, fc1b,
      fc2w.astype(_CDT), fc2b)
    return out[:, :10]


# speedup vs baseline: 1.0047x; 1.0047x over previous
import numpy as np
import jax
import jax.numpy as jnp
from jax.experimental import pallas as pl
from jax.experimental.pallas import tpu as pltpu

_CDT = jnp.bfloat16


def _w1_sel():
    s = np.zeros((5, 5, 6, 28, 4, 12), np.float32)
    for rp in range(2):
        for wp in range(2):
            g = rp * 2 + wp
            for i in range(5):
                for j in range(5):
                    for ow in range(12):
                        s[i, j, i + rp, 2 * ow + wp + j, g, ow] = 1.0
    return s.reshape(25, 6 * 28 * 4 * 12)


def _w2_sel():
    s = np.zeros((5, 12, 2, 4), np.float32)
    for wp in range(2):
        for j in range(5):
            for o2 in range(4):
                s[j, 2 * o2 + wp + j, wp, o2] = 1.0
    return s.reshape(5, 96)


_W1_SEL = _w1_sel()
_W2_SEL = _w2_sel()


def _net_kernel(x_ref, w1_ref, b1_ref, w2_ref, b2_ref, fs_ref, fb1_ref,
                fw2_ref, fb2_ref, o_ref):
    G = o_ref.shape[0]
    f32 = jnp.float32
    xb = x_ref[...]

    w1 = w1_ref[...]
    b1 = b1_ref[...]
    pieces = []
    for hp in range(12):
        c1 = jnp.dot(xb[:, 56 * hp:56 * hp + 168], w1,
                     preferred_element_type=f32)
        m = jnp.maximum(jnp.maximum(c1[:, 0:256], c1[:, 256:512]),
                        jnp.maximum(c1[:, 512:768], c1[:, 768:1024]))
        y = jnp.maximum(m + b1, 0.0)
        pieces.append(y.astype(_CDT))
    y1 = jnp.concatenate(pieces, axis=1)

    w2 = w2_ref[...]
    b2 = b2_ref[...]
    pm = []
    for oh in range(8):
        c2 = jnp.dot(y1[:, 256 * oh:256 * oh + 1280], w2,
                     preferred_element_type=f32)
        pm.append(jnp.maximum(c2[:, 0:256], c2[:, 256:512]))

    h = jnp.zeros((G, 512), f32)
    for r in range(4):
        y2 = jnp.maximum(jnp.maximum(pm[2 * r], pm[2 * r + 1]) + b2, 0.0)
        h = h + jnp.dot(y2.astype(_CDT), fs_ref[r],
                        preferred_element_type=f32)

    hr = jnp.maximum(h + fb1_ref[...], 0.0).astype(_CDT)
    z = jnp.dot(hr, fw2_ref[...], preferred_element_type=f32) + fb2_ref[...]
    mz = jnp.max(z, axis=-1, keepdims=True)
    ez = jnp.exp(z - mz)
    lse = jnp.log(jnp.sum(ez, axis=-1, keepdims=True)) + mz
    o_ref[...] = (z - lse)[:, :16]


def kernel(x, w1, b1, w2, b2, se1, so1, s2, fc1w, fc1b, fc2w, fc2b):
    del se1, so1, s2
    B = x.shape[0]
    G = 512
    while B % G:
        G //= 2

    w1all = jnp.einsum("tc,tm->mc", w1[:, :20], _W1_SEL)
    w1all = w1all.reshape(168, 4, 240)
    w1all = jnp.pad(w1all, ((0, 0), (0, 0), (0, 16)))
    w1all = w1all.reshape(168, 1024).astype(_CDT)
    b1r = jnp.pad(jnp.tile(b1[0, :20], 12), (0, 16)).reshape(1, 256)
    w2c = w2[:, :20, :50].reshape(5, 5, 20, 50)
    w2s = jnp.einsum("ijab,jm->imab", w2c, _W2_SEL)
    w2s = w2s.reshape(5, 12, 2, 4, 20, 50).transpose(0, 1, 4, 2, 3, 5)
    w2s = w2s.reshape(5, 240, 2, 200)
    w2s = jnp.pad(w2s, ((0, 0), (0, 16), (0, 0), (0, 56)))
    w2big = w2s.reshape(1280, 512).astype(_CDT)
    b2r = jnp.pad(jnp.tile(b2[0, :50], 4), (0, 56)).reshape(1, 256)

    f3 = fc1w.reshape(16, 128, 512)[:, :50, :].reshape(4, 200, 512)
    fs = jnp.pad(f3, ((0, 0), (0, 56), (0, 0))).astype(_CDT)

    out = pl.pallas_call(
        _net_kernel,
        grid=(B // G,),
        out_shape=jax.ShapeDtypeStruct((B, 16), jnp.float32),
        in_specs=[
            pl.BlockSpec((G, 784), lambda b: (b, 0)),
            pl.BlockSpec((168, 1024), lambda b: (0, 0)),
            pl.BlockSpec((1, 256), lambda b: (0, 0)),
            pl.BlockSpec((1280, 512), lambda b: (0, 0)),
            pl.BlockSpec((1, 256), lambda b: (0, 0)),
            pl.BlockSpec((4, 256, 512), lambda b: (0, 0, 0)),
            pl.BlockSpec((1, 512), lambda b: (0, 0)),
            pl.BlockSpec((512, 128), lambda b: (0, 0)),
            pl.BlockSpec((1, 128), lambda b: (0, 0)),
        ],
        out_specs=pl.BlockSpec((G, 16), lambda b: (b, 0)),
        compiler_params=pltpu.CompilerParams(
            dimension_semantics=("parallel",)),
    )(x.reshape(B, 784).astype(_CDT), w1all, b1r, w2big, b2r, fs, fc1b,
      fc2w.astype(_CDT), fc2b)
    return out[:, :10]
